# Initial kernel scaffold; baseline (speedup 1.0000x reference)
import jax
import jax.numpy as jnp
from jax.experimental import pallas as pl


def kernel(edge_index, logits, scale):
    raise NotImplementedError("write your pallas kernel here")


# speedup vs baseline: 1.5992x; 1.5992x over previous
import functools

import jax
import jax.numpy as jnp
from jax import lax
from jax.experimental import pallas as pl
from jax.experimental.pallas import tpu as pltpu
from jax.experimental.pallas import tpu_sc as plsc

N_NODES = 50000
LANES = 16
N_PAD = 50048
QUARTER = N_PAD // 4
SUBQ = QUARTER // 2
CHUNK = 8000


def _sc_body(E, EP, NCHUNK,
             dst_hbm, lgT_hbm, scT_hbm,
             out_hbm, part_hbm, fin_hbm,
             table, dst_buf, lg_buf, sc_buf, out_buf):
    c = lax.axis_index("c")
    s = lax.axis_index("s")
    head_local = s // 4
    head = c * 4 + head_local
    part = s % 4
    w = c * 16 + s

    def init_body(i, _):
        table[pl.ds(i * LANES, LANES)] = jnp.full((LANES,), -jnp.inf, jnp.float32)
        return 0
    lax.fori_loop(0, N_PAD // LANES, init_body, 0)

    def chunk1(ci, _):
        base = pl.multiple_of(part * EP + ci * CHUNK, 8)
        pltpu.sync_copy(dst_hbm.at[pl.ds(base, CHUNK)], dst_buf)
        pltpu.sync_copy(lgT_hbm.at[pl.ds(head * E + base, CHUNK)], lg_buf)

        def vec(i, _):
            d = dst_buf[pl.ds(i * LANES, LANES)]
            v = lg_buf[pl.ds(i * LANES, LANES)]
            g = plsc.load_gather(table, [d])

            def cond(gc):
                return jnp.any(v > gc)

            def wbody(gc):
                plsc.store_scatter(table, [d], v, mask=v > gc)
                return plsc.load_gather(table, [d])

            lax.while_loop(cond, wbody, g)
            return 0
        lax.fori_loop(0, CHUNK // LANES, vec, 0)
        return 0
    lax.fori_loop(0, NCHUNK, chunk1, 0)

    pltpu.sync_copy(table, part_hbm.at[pl.ds(w * N_PAD, N_PAD)])
    plsc.subcore_barrier()

    team = c * 16 + head_local * 4
    for q2 in range(2):
        qoff = part * QUARTER + q2 * SUBQ
        pltpu.sync_copy(part_hbm.at[pl.ds(team * N_PAD + qoff, SUBQ)],
                        lg_buf.at[pl.ds(0, SUBQ)])
        for j in range(1, 4):
            pltpu.sync_copy(part_hbm.at[pl.ds((team + j) * N_PAD + qoff, SUBQ)],
                            sc_buf.at[pl.ds(0, SUBQ)])

            def mx_body(i, _):
                sl = pl.ds(i * LANES, LANES)
                lg_buf[sl] = jnp.maximum(lg_buf[sl], sc_buf[sl])
                return 0
            lax.fori_loop(0, SUBQ // LANES, mx_body, 0)
        pltpu.sync_copy(lg_buf.at[pl.ds(0, SUBQ)],
                        fin_hbm.at[pl.ds(head * N_PAD + qoff, SUBQ)])
    plsc.subcore_barrier()
    pltpu.sync_copy(fin_hbm.at[pl.ds(head * N_PAD, N_PAD)], table)

    def chunk3(ci, _):
        base = pl.multiple_of(part * EP + ci * CHUNK, 8)
        pltpu.sync_copy(dst_hbm.at[pl.ds(base, CHUNK)], dst_buf)
        pltpu.sync_copy(lgT_hbm.at[pl.ds(head * E + base, CHUNK)], lg_buf)
        pltpu.sync_copy(scT_hbm.at[pl.ds(head * E + base, CHUNK)], sc_buf)

        def vec(i, _):
            sl = pl.ds(i * LANES, LANES)
            d = dst_buf[sl]
            mx = plsc.load_gather(table, [d])
            out_buf[sl] = sc_buf[sl] * jnp.exp(lg_buf[sl] - mx)
            return 0
        lax.fori_loop(0, CHUNK // LANES, vec, 0)
        pltpu.sync_copy(out_buf, out_hbm.at[pl.ds(head * E + base, CHUNK)])
        return 0
    lax.fori_loop(0, NCHUNK, chunk3, 0)


def kernel(edge_index, logits, scale):
    E, H = scale.shape
    assert H == 8 and E % (4 * CHUNK) == 0
    EP = E // 4
    NCHUNK = EP // CHUNK

    dst = edge_index[1]
    lgT = logits.reshape(E, H).T.reshape(-1)
    scT = scale.T.reshape(-1)

    mesh = plsc.VectorSubcoreMesh(core_axis_name="c", subcore_axis_name="s")
    body = functools.partial(_sc_body, E, EP, NCHUNK)
    eT, _parts, _fin = pl.kernel(
        body,
        out_type=(
            jax.ShapeDtypeStruct((H * E,), jnp.float32),
            jax.ShapeDtypeStruct((32 * N_PAD,), jnp.float32),
            jax.ShapeDtypeStruct((8 * N_PAD,), jnp.float32),
        ),
        mesh=mesh,
        compiler_params=pltpu.CompilerParams(needs_layout_passes=False),
        scratch_types=[
            pltpu.VMEM((N_PAD,), jnp.float32),
            pltpu.VMEM((CHUNK,), jnp.int32),
            pltpu.VMEM((CHUNK,), jnp.float32),
            pltpu.VMEM((CHUNK,), jnp.float32),
            pltpu.VMEM((CHUNK,), jnp.float32),
        ],
    )(dst, lgT, scT)

    return eT.reshape(H, E).T.reshape(E, H, 1)
